# baseline (speedup 1.0000x reference)
import functools

import jax
import jax.numpy as jnp
from jax import lax
from jax.experimental import pallas as pl
from jax.experimental.pallas import tpu as pltpu
from jax.experimental.pallas import tpu_sc as plsc

N_NODES = 50000
N_EDGES = 800000
MSG = 10
HID = 10
NUM_ET = 8
NUM_CLASSES = 16

MSGP = 16
ROW_BLK = 2048
N_GRID = (N_NODES + ROW_BLK - 1) // ROW_BLK

NW = 32
IDX_W = 128
STREAMS_PER_STEP = 10
STEP_E = IDX_W * STREAMS_PER_STEP
STEPS_PER_W = 20
E_PER_W = STEP_E * STEPS_PER_W
E_PAD = E_PER_W * NW
ROWS_PER_W = E_PER_W // IDX_W
CORE0_ROWS = 400
CORE1_ROWS = 0
NPAD = 50048
NPW = NPAD // 16


PACK_BLK = ROW_BLK // 8
N_PACK = N_NODES // 8
NPAD_PACK = NPAD * MSGP // 128


def _dense_pre_body(f_ref, fp_ref, t_ref, bdhr, bdhz, bdhn, bhr, bhz, bhn,
                    x_ref, ghr_ref, ghz_ref, ghn_ref):
    f = f_ref[...]
    fp = fp_ref[...]
    x_ref[...] = jnp.dot(f, t_ref[...], preferred_element_type=jnp.float32)
    ghr_ref[...] = jnp.dot(fp, bdhr[...], preferred_element_type=jnp.float32) + bhr[...]
    ghz_ref[...] = jnp.dot(fp, bdhz[...], preferred_element_type=jnp.float32) + bhz[...]
    ghn_ref[...] = jnp.dot(fp, bdhn[...], preferred_element_type=jnp.float32) + bhn[...]


def _dense_pre(features, fp, t2, bdhr, bdhz, bdhn, bhr, bhz, bhn):
    row = lambda i: (i, 0)
    fixed = lambda i: (0, 0)
    packed = jax.ShapeDtypeStruct((N_PACK, 128), jnp.float32)
    pblk = pl.BlockSpec((PACK_BLK, 128), row)
    return pl.pallas_call(
        _dense_pre_body,
        grid=(N_GRID,),
        in_specs=[
            pl.BlockSpec((ROW_BLK, HID), row),
            pblk,
            pl.BlockSpec((HID, NUM_ET * MSGP), fixed),
            pl.BlockSpec((128, 128), fixed),
            pl.BlockSpec((128, 128), fixed),
            pl.BlockSpec((128, 128), fixed),
            pl.BlockSpec((1, 128), fixed),
            pl.BlockSpec((1, 128), fixed),
            pl.BlockSpec((1, 128), fixed),
        ],
        out_specs=[
            pl.BlockSpec((ROW_BLK, NUM_ET * MSGP), row),
            pblk, pblk, pblk,
        ],
        out_shape=[
            jax.ShapeDtypeStruct((N_NODES, NUM_ET * MSGP), jnp.float32),
            packed, packed, packed,
        ],
    )(features, fp, t2, bdhr, bdhz, bdhn, bhr, bhz, bhn)


@functools.cache
def _make_sc_kernel():
    @functools.partial(
        pl.kernel,
        out_type=jax.ShapeDtypeStruct((2, NPAD, MSGP), jnp.float32),
        mesh=plsc.VectorSubcoreMesh(core_axis_name="c", subcore_axis_name="s"),
        compiler_params=pltpu.CompilerParams(use_tc_tiling_on_sc=False),
        scratch_types=[
            pltpu.VMEM((4, STREAMS_PER_STEP, IDX_W), jnp.int32),
            pltpu.VMEM((4, STREAMS_PER_STEP, IDX_W), jnp.int32),
            pltpu.VMEM((2, STEP_E, MSGP), jnp.float32),
            pltpu.VMEM_SHARED((NPAD, MSGP), jnp.float32),
            pltpu.SemaphoreType.DMA,
            pltpu.SemaphoreType.DMA,
            pltpu.SemaphoreType.DMA,
        ],
    )
    def _sc_gather_scatter(zeros_hbm, gidx_hbm, didx_hbm, x2_hbm, out_hbm,
                           gv, dv, rows, acc, sem_i, sem_g, sem_s):
        cid = lax.axis_index("c")
        sid = lax.axis_index("s")
        my_rows = lax.select(cid == 0, CORE0_ROWS, CORE1_ROWS)
        my_steps = my_rows // STREAMS_PER_STEP
        row_base = cid * 16 * CORE0_ROWS + sid * my_rows
        r0 = sid * NPW
        pltpu.sync_copy(zeros_hbm.at[pl.ds(r0, NPW)], acc.at[pl.ds(r0, NPW)])
        plsc.subcore_barrier()

        def idx_copies(bb, b4):
            base = row_base + bb * STREAMS_PER_STEP
            src = pl.ds(base, STREAMS_PER_STEP)
            return (pltpu.make_async_copy(gidx_hbm.at[src], gv.at[b4], sem_i),
                    pltpu.make_async_copy(didx_hbm.at[src], dv.at[b4], sem_i))

        def scat_desc(b2, b4, r):
            return pltpu.make_async_copy(
                rows.at[b2, pl.ds(r * IDX_W, IDX_W)],
                acc.at[dv.at[b4, r]], sem_s)

        for pb in (0, 1):
            @pl.when(pb < my_steps)
            def _():
                for c in idx_copies(pb, pb):
                    c.start()

        def step(bb, carry):
            b2 = lax.rem(bb, 2)
            b4 = lax.rem(bb, 4)
            nb4 = lax.rem(bb + 2, 4)
            for c in idx_copies(bb, b4):
                c.wait()
            @pl.when(bb >= 2)
            def _():
                for r in range(STREAMS_PER_STEP):
                    scat_desc(b2, nb4, r).wait()
            gcps = [
                pltpu.async_copy(x2_hbm.at[gv.at[b4, r]],
                                 rows.at[b2, pl.ds(r * IDX_W, IDX_W)], sem_g)
                for r in range(STREAMS_PER_STEP)
            ]
            @pl.when(bb < my_steps - 2)
            def _():
                for c in idx_copies(bb + 2, nb4):
                    c.start()
            for c in gcps:
                c.wait()
            for r in range(STREAMS_PER_STEP):
                pltpu.async_copy(rows.at[b2, pl.ds(r * IDX_W, IDX_W)],
                                 acc.at[dv.at[b4, r]], sem_s, add=True)
            return carry

        lax.fori_loop(0, my_steps, step, 0)
        for db in (2, 1):
            bb = my_steps - db

            @pl.when(bb >= 0)
            def _():
                for r in range(STREAMS_PER_STEP):
                    scat_desc(lax.rem(bb, 2), lax.rem(bb, 4), r).wait()
        plsc.subcore_barrier()
        pltpu.sync_copy(acc.at[pl.ds(r0, NPW)], out_hbm.at[cid, pl.ds(r0, NPW)])

    return _sc_gather_scatter


def _dense_post_body(pa_ref, pb_ref, ghr_ref, ghz_ref, ghn_ref, fp_ref,
                     bdr, bdz, bdn, bir, biz, bin_, bdout, bout, out_ref):
    red = pa_ref[0] + pb_ref[0]
    gi_r = jnp.dot(red, bdr[...], preferred_element_type=jnp.float32) + bir[...]
    gi_z = jnp.dot(red, bdz[...], preferred_element_type=jnp.float32) + biz[...]
    gi_n = jnp.dot(red, bdn[...], preferred_element_type=jnp.float32) + bin_[...]
    r = jax.nn.sigmoid(gi_r + ghr_ref[...])
    z = jax.nn.sigmoid(gi_z + ghz_ref[...])
    n = jnp.tanh(gi_n + r * ghn_ref[...])
    h = (1.0 - z) * n + z * fp_ref[...]
    out_ref[...] = jnp.dot(h, bdout[...], preferred_element_type=jnp.float32) + bout[...]


def _dense_post(pp, ghr, ghz, ghn, fp,
                bdr, bdz, bdn, bir, biz, bin_, bdout, bout):
    prow = lambda i: (i, 0)
    fixed = lambda i: (0, 0)
    pblk = pl.BlockSpec((PACK_BLK, 128), prow)
    return pl.pallas_call(
        _dense_post_body,
        grid=(N_GRID,),
        in_specs=[
            pl.BlockSpec((1, PACK_BLK, 128), lambda i: (0, i, 0)),
            pl.BlockSpec((1, PACK_BLK, 128), lambda i: (1, i, 0)),
            pblk, pblk, pblk, pblk,
            pl.BlockSpec((128, 128), fixed),
            pl.BlockSpec((128, 128), fixed),
            pl.BlockSpec((128, 128), fixed),
            pl.BlockSpec((1, 128), fixed),
            pl.BlockSpec((1, 128), fixed),
            pl.BlockSpec((1, 128), fixed),
            pl.BlockSpec((128, 128), fixed),
            pl.BlockSpec((1, 128), fixed),
        ],
        out_specs=pl.BlockSpec((PACK_BLK, 128), prow),
        out_shape=jax.ShapeDtypeStruct((N_PACK, 128), jnp.float32),
    )(pp, pp, ghr, ghz, ghn, fp,
      bdr, bdz, bdn, bir, biz, bin_, bdout, bout)


def kernel(features, src_list, dst_list, edge_types, edge_matrix,
           W_ih, W_hh, b_ih, b_hh, W_out, b_out):
    f32 = jnp.float32
    em3 = edge_matrix.reshape(NUM_ET, MSG, HID).transpose(2, 0, 1)
    t2 = jnp.pad(em3, ((0, 0), (0, 0), (0, MSGP - MSG))).reshape(HID, NUM_ET * MSGP)

    eye8 = jnp.eye(8, dtype=f32)
    pad_sq = lambda w: jnp.pad(w, ((0, MSGP - w.shape[0]), (0, MSGP - w.shape[1])))
    bd = lambda w: jnp.kron(eye8, pad_sq(w))
    tile_b = lambda b: jnp.tile(jnp.pad(b, (0, MSGP - b.shape[0])), 8).reshape(1, 128)
    bdhr = bd(W_hh[0 * HID:1 * HID].T)
    bdhz = bd(W_hh[1 * HID:2 * HID].T)
    bdhn = bd(W_hh[2 * HID:3 * HID].T)
    bhr = tile_b(b_hh[0 * HID:1 * HID])
    bhz = tile_b(b_hh[1 * HID:2 * HID])
    bhn = tile_b(b_hh[2 * HID:3 * HID])
    bdr = bd(W_ih[0 * HID:1 * HID].T)
    bdz = bd(W_ih[1 * HID:2 * HID].T)
    bdn = bd(W_ih[2 * HID:3 * HID].T)
    bir = tile_b(b_ih[0 * HID:1 * HID])
    biz = tile_b(b_ih[1 * HID:2 * HID])
    bin_ = tile_b(b_ih[2 * HID:3 * HID])

    bdout = bd(W_out.T)
    bout = jnp.tile(b_out, 8).reshape(1, 128)

    fp = jnp.pad(features, ((0, 0), (0, MSGP - HID))).reshape(N_PACK, 128)

    x, ghr, ghz, ghn = _dense_pre(features, fp, t2,
                                  bdhr, bdhz, bdhn, bhr, bhz, bhn)
    x2 = x.reshape(N_NODES * NUM_ET, MSGP)

    pad_e = E_PAD - N_EDGES
    gidx = src_list * NUM_ET + edge_types
    gidx = jnp.concatenate([gidx, jnp.zeros((pad_e,), jnp.int32)])
    didx = jnp.concatenate([dst_list, jnp.full((pad_e,), N_NODES, jnp.int32)])
    gidx2 = gidx.reshape(-1, IDX_W)
    didx2 = didx.reshape(-1, IDX_W)
    zeros_acc = jnp.zeros((NPAD, MSGP), f32)

    partials = _make_sc_kernel()(zeros_acc, gidx2, didx2, x2)
    pp = partials.reshape(2, NPAD_PACK, 128)

    out_p = _dense_post(pp, ghr, ghz, ghn, fp,
                        bdr, bdz, bdn, bir, biz, bin_, bdout, bout)
    return out_p.reshape(N_NODES, NUM_CLASSES)


# speedup vs baseline: 1.2483x; 1.2483x over previous
import functools

import jax
import jax.numpy as jnp
from jax import lax
from jax.experimental import pallas as pl
from jax.experimental.pallas import tpu as pltpu
from jax.experimental.pallas import tpu_sc as plsc

N_NODES = 50000
N_EDGES = 800000
MSG = 10
HID = 10
NUM_ET = 8
NUM_CLASSES = 16

MSGP = 16
ROW_BLK = 2048
N_GRID = (N_NODES + ROW_BLK - 1) // ROW_BLK

NW = 32
IDX_W = 128
STREAMS_PER_STEP = 10
STEP_E = IDX_W * STREAMS_PER_STEP
STEPS_PER_W = 20
E_PER_W = STEP_E * STEPS_PER_W
E_PAD = E_PER_W * NW
ROWS_PER_W = E_PER_W // IDX_W
CORE0_ROWS = 280
CORE1_ROWS = 120
NPAD = 50048
NPW = NPAD // 16


PACK_BLK = ROW_BLK // 8
N_PACK = N_NODES // 8
NPAD_PACK = NPAD * MSGP // 128


def _x_table_body(f_ref, t_ref, x_ref):
    x_ref[...] = jnp.dot(f_ref[...], t_ref[...],
                         preferred_element_type=jnp.float32)


def _x_table(features, t2):
    row = lambda i: (i, 0)
    fixed = lambda i: (0, 0)
    return pl.pallas_call(
        _x_table_body,
        grid=(N_GRID,),
        in_specs=[
            pl.BlockSpec((ROW_BLK, HID), row),
            pl.BlockSpec((HID, NUM_ET * MSGP), fixed),
        ],
        out_specs=pl.BlockSpec((ROW_BLK, NUM_ET * MSGP), row),
        out_shape=jax.ShapeDtypeStruct((N_NODES, NUM_ET * MSGP), jnp.float32),
    )(features, t2)


def _h_gates_body(fp_ref, bdhr, bdhz, bdhn, bhr, bhz, bhn,
                  ghr_ref, ghz_ref, ghn_ref):
    fp = fp_ref[...]
    ghr_ref[...] = jnp.dot(fp, bdhr[...], preferred_element_type=jnp.float32) + bhr[...]
    ghz_ref[...] = jnp.dot(fp, bdhz[...], preferred_element_type=jnp.float32) + bhz[...]
    ghn_ref[...] = jnp.dot(fp, bdhn[...], preferred_element_type=jnp.float32) + bhn[...]


def _h_gates(fp, bdhr, bdhz, bdhn, bhr, bhz, bhn):
    row = lambda i: (i, 0)
    fixed = lambda i: (0, 0)
    packed = jax.ShapeDtypeStruct((N_PACK, 128), jnp.float32)
    pblk = pl.BlockSpec((PACK_BLK, 128), row)
    return pl.pallas_call(
        _h_gates_body,
        grid=(N_GRID,),
        in_specs=[
            pblk,
            pl.BlockSpec((128, 128), fixed),
            pl.BlockSpec((128, 128), fixed),
            pl.BlockSpec((128, 128), fixed),
            pl.BlockSpec((1, 128), fixed),
            pl.BlockSpec((1, 128), fixed),
            pl.BlockSpec((1, 128), fixed),
        ],
        out_specs=[pblk, pblk, pblk],
        out_shape=[packed, packed, packed],
    )(fp, bdhr, bdhz, bdhn, bhr, bhz, bhn)


@functools.cache
def _make_sc_kernel():
    @functools.partial(
        pl.kernel,
        out_type=jax.ShapeDtypeStruct((2, NPAD, MSGP), jnp.float32),
        mesh=plsc.VectorSubcoreMesh(core_axis_name="c", subcore_axis_name="s"),
        compiler_params=pltpu.CompilerParams(use_tc_tiling_on_sc=False),
        scratch_types=[
            pltpu.VMEM((4, STREAMS_PER_STEP, IDX_W), jnp.int32),
            pltpu.VMEM((4, STREAMS_PER_STEP, IDX_W), jnp.int32),
            pltpu.VMEM((2, STEP_E, MSGP), jnp.float32),
            pltpu.VMEM_SHARED((NPAD, MSGP), jnp.float32),
            pltpu.SemaphoreType.DMA,
            pltpu.SemaphoreType.DMA,
            pltpu.SemaphoreType.DMA,
        ],
    )
    def _sc_gather_scatter(zeros_hbm, gidx_hbm, didx_hbm, x2_hbm, out_hbm,
                           gv, dv, rows, acc, sem_i, sem_g, sem_s):
        cid = lax.axis_index("c")
        sid = lax.axis_index("s")
        my_rows = lax.select(cid == 0, CORE0_ROWS, CORE1_ROWS)
        my_steps = my_rows // STREAMS_PER_STEP
        row_base = cid * 16 * CORE0_ROWS + sid * my_rows
        r0 = sid * NPW
        pltpu.sync_copy(zeros_hbm.at[pl.ds(r0, NPW)], acc.at[pl.ds(r0, NPW)])
        plsc.subcore_barrier()

        def idx_copies(bb, b4):
            base = row_base + bb * STREAMS_PER_STEP
            src = pl.ds(base, STREAMS_PER_STEP)
            return (pltpu.make_async_copy(gidx_hbm.at[src], gv.at[b4], sem_i),
                    pltpu.make_async_copy(didx_hbm.at[src], dv.at[b4], sem_i))

        def scat_desc(b2, b4, r):
            return pltpu.make_async_copy(
                rows.at[b2, pl.ds(r * IDX_W, IDX_W)],
                acc.at[dv.at[b4, r]], sem_s)

        for pb in (0, 1):
            @pl.when(pb < my_steps)
            def _():
                for c in idx_copies(pb, pb):
                    c.start()

        def step(bb, carry):
            b2 = lax.rem(bb, 2)
            b4 = lax.rem(bb, 4)
            nb4 = lax.rem(bb + 2, 4)
            for c in idx_copies(bb, b4):
                c.wait()
            @pl.when(bb >= 2)
            def _():
                for r in range(STREAMS_PER_STEP):
                    scat_desc(b2, nb4, r).wait()
            gcps = [
                pltpu.async_copy(x2_hbm.at[gv.at[b4, r]],
                                 rows.at[b2, pl.ds(r * IDX_W, IDX_W)], sem_g)
                for r in range(STREAMS_PER_STEP)
            ]
            @pl.when(bb < my_steps - 2)
            def _():
                for c in idx_copies(bb + 2, nb4):
                    c.start()
            for c in gcps:
                c.wait()
            for r in range(STREAMS_PER_STEP):
                pltpu.async_copy(rows.at[b2, pl.ds(r * IDX_W, IDX_W)],
                                 acc.at[dv.at[b4, r]], sem_s, add=True)
            return carry

        lax.fori_loop(0, my_steps, step, 0)
        for db in (2, 1):
            bb = my_steps - db

            @pl.when(bb >= 0)
            def _():
                for r in range(STREAMS_PER_STEP):
                    scat_desc(lax.rem(bb, 2), lax.rem(bb, 4), r).wait()
        plsc.subcore_barrier()
        pltpu.sync_copy(acc.at[pl.ds(r0, NPW)], out_hbm.at[cid, pl.ds(r0, NPW)])

    return _sc_gather_scatter


def _dense_post_body(pa_ref, pb_ref, ghr_ref, ghz_ref, ghn_ref, fp_ref,
                     bdr, bdz, bdn, bir, biz, bin_, bdout, bout, out_ref):
    red = pa_ref[0] + pb_ref[0]
    gi_r = jnp.dot(red, bdr[...], preferred_element_type=jnp.float32) + bir[...]
    gi_z = jnp.dot(red, bdz[...], preferred_element_type=jnp.float32) + biz[...]
    gi_n = jnp.dot(red, bdn[...], preferred_element_type=jnp.float32) + bin_[...]
    r = jax.nn.sigmoid(gi_r + ghr_ref[...])
    z = jax.nn.sigmoid(gi_z + ghz_ref[...])
    n = jnp.tanh(gi_n + r * ghn_ref[...])
    h = (1.0 - z) * n + z * fp_ref[...]
    out_ref[...] = jnp.dot(h, bdout[...], preferred_element_type=jnp.float32) + bout[...]


def _dense_post(pp, ghr, ghz, ghn, fp,
                bdr, bdz, bdn, bir, biz, bin_, bdout, bout):
    prow = lambda i: (i, 0)
    fixed = lambda i: (0, 0)
    pblk = pl.BlockSpec((PACK_BLK, 128), prow)
    return pl.pallas_call(
        _dense_post_body,
        grid=(N_GRID,),
        in_specs=[
            pl.BlockSpec((1, PACK_BLK, 128), lambda i: (0, i, 0)),
            pl.BlockSpec((1, PACK_BLK, 128), lambda i: (1, i, 0)),
            pblk, pblk, pblk, pblk,
            pl.BlockSpec((128, 128), fixed),
            pl.BlockSpec((128, 128), fixed),
            pl.BlockSpec((128, 128), fixed),
            pl.BlockSpec((1, 128), fixed),
            pl.BlockSpec((1, 128), fixed),
            pl.BlockSpec((1, 128), fixed),
            pl.BlockSpec((128, 128), fixed),
            pl.BlockSpec((1, 128), fixed),
        ],
        out_specs=pl.BlockSpec((PACK_BLK, 128), prow),
        out_shape=jax.ShapeDtypeStruct((N_PACK, 128), jnp.float32),
    )(pp, pp, ghr, ghz, ghn, fp,
      bdr, bdz, bdn, bir, biz, bin_, bdout, bout)


def kernel(features, src_list, dst_list, edge_types, edge_matrix,
           W_ih, W_hh, b_ih, b_hh, W_out, b_out):
    f32 = jnp.float32
    em3 = edge_matrix.reshape(NUM_ET, MSG, HID).transpose(2, 0, 1)
    t2 = jnp.pad(em3, ((0, 0), (0, 0), (0, MSGP - MSG))).reshape(HID, NUM_ET * MSGP)

    eye8 = jnp.eye(8, dtype=f32)
    pad_sq = lambda w: jnp.pad(w, ((0, MSGP - w.shape[0]), (0, MSGP - w.shape[1])))
    bd = lambda w: jnp.kron(eye8, pad_sq(w))
    tile_b = lambda b: jnp.tile(jnp.pad(b, (0, MSGP - b.shape[0])), 8).reshape(1, 128)
    bdhr = bd(W_hh[0 * HID:1 * HID].T)
    bdhz = bd(W_hh[1 * HID:2 * HID].T)
    bdhn = bd(W_hh[2 * HID:3 * HID].T)
    bhr = tile_b(b_hh[0 * HID:1 * HID])
    bhz = tile_b(b_hh[1 * HID:2 * HID])
    bhn = tile_b(b_hh[2 * HID:3 * HID])
    bdr = bd(W_ih[0 * HID:1 * HID].T)
    bdz = bd(W_ih[1 * HID:2 * HID].T)
    bdn = bd(W_ih[2 * HID:3 * HID].T)
    bir = tile_b(b_ih[0 * HID:1 * HID])
    biz = tile_b(b_ih[1 * HID:2 * HID])
    bin_ = tile_b(b_ih[2 * HID:3 * HID])

    bdout = bd(W_out.T)
    bout = jnp.tile(b_out, 8).reshape(1, 128)

    fp = jnp.pad(features, ((0, 0), (0, MSGP - HID))).reshape(N_PACK, 128)

    x = _x_table(features, t2)
    x2 = x.reshape(N_NODES * NUM_ET, MSGP)
    ghr, ghz, ghn = _h_gates(fp, bdhr, bdhz, bdhn, bhr, bhz, bhn)

    pad_e = E_PAD - N_EDGES
    gidx = src_list * NUM_ET + edge_types
    gidx = jnp.concatenate([gidx, jnp.zeros((pad_e,), jnp.int32)])
    didx = jnp.concatenate([dst_list, jnp.full((pad_e,), N_NODES, jnp.int32)])
    gidx2 = gidx.reshape(-1, IDX_W)
    didx2 = didx.reshape(-1, IDX_W)
    zeros_acc = jnp.zeros((NPAD, MSGP), f32)

    partials = _make_sc_kernel()(zeros_acc, gidx2, didx2, x2)
    pp = partials.reshape(2, NPAD_PACK, 128)

    out_p = _dense_post(pp, ghr, ghz, ghn, fp,
                        bdr, bdz, bdn, bir, biz, bin_, bdout, bout)
    return out_p.reshape(N_NODES, NUM_CLASSES)
